# Initial kernel scaffold; baseline (speedup 1.0000x reference)
import jax
import jax.numpy as jnp
from jax.experimental import pallas as pl


def kernel(t, h, x, edge_index, params):
    raise NotImplementedError("write your pallas kernel here")


# speedup vs baseline: 17.2480x; 17.2480x over previous
import functools
import jax
import jax.numpy as jnp
from jax import lax
from jax.experimental import pallas as pl
from jax.experimental.pallas import tpu as pltpu
from jax.experimental.pallas import tpu_sc as plsc

NC, NS = 2, 16
NW = NC * NS
CR = 10.0


@functools.lru_cache(maxsize=None)
def _sc_gather(E, D, C):
    EW = E // NW
    assert E % NW == 0 and EW % C == 0 and C % 8 == 0
    nit = EW // C
    mesh = plsc.VectorSubcoreMesh(core_axis_name="c", subcore_axis_name="s")

    def body(table, idx, out, idx_v, rows_v, sem):
        wid = lax.axis_index("s") * NC + lax.axis_index("c")
        base = wid * EW

        def step(i, _):
            off = base + i * C
            pltpu.sync_copy(idx.at[pl.ds(off, C)], idx_v)
            pltpu.async_copy(table.at[idx_v], rows_v, sem).wait()
            pltpu.sync_copy(rows_v, out.at[pl.ds(off, C)])
            return 0

        lax.fori_loop(0, nit, step, 0)

    def run(table, idx):
        return pl.kernel(
            body,
            out_type=jax.ShapeDtypeStruct((E, D), jnp.float32),
            mesh=mesh,
            compiler_params=pltpu.CompilerParams(use_tc_tiling_on_sc=False),
            scratch_types=[
                pltpu.VMEM((C,), jnp.int32),
                pltpu.VMEM((C, D), jnp.float32),
                pltpu.SemaphoreType.DMA,
            ],
        )(table, idx)

    return run


def _gather(table, idx):
    table = jnp.asarray(table, jnp.float32)
    E = idx.shape[0]
    D = table.shape[1]
    return _sc_gather(E, D, 1000)(table, idx)


@functools.lru_cache(maxsize=None)
def _sc_scatter(E, D, Npad, C):
    assert D % 8 == 0
    chunks = []
    c0 = 0
    while c0 < D:
        dc = min(16, D - c0)
        chunks.append((c0, dc))
        c0 += dc
    half = (len(chunks) + 1) // 2
    per_core = [tuple(chunks[:half]), tuple(chunks[half:])]
    EW = E // NS
    assert E % NS == 0 and EW % C == 0 and C % 8 == 0
    nit = EW // C
    RPT = Npad // NS
    assert Npad % NS == 0 and RPT % 8 == 0
    mesh = plsc.VectorSubcoreMesh(core_axis_name="c", subcore_axis_name="s")

    def body(vals, idx, zeros, out, idx_v, val_v, acc, sem):
        cc = lax.axis_index("c")
        sid = lax.axis_index("s")
        pltpu.sync_copy(zeros.at[pl.ds(0, C)], val_v)

        def do_group(g0, gD):
            pltpu.sync_copy(zeros.at[pl.ds(sid * RPT, RPT)],
                            acc.at[pl.ds(sid * RPT, RPT)])
            plsc.subcore_barrier()

            def step(i, _):
                off = sid * EW + i * C
                pltpu.sync_copy(idx.at[pl.ds(off, C)], idx_v)
                pltpu.sync_copy(vals.at[pl.ds(off, C), pl.ds(g0, gD)],
                                val_v.at[pl.ds(0, C), pl.ds(0, gD)])
                pltpu.sync_copy(val_v, acc.at[idx_v], add=True)
                return 0

            lax.fori_loop(0, nit, step, 0)
            plsc.subcore_barrier()
            pltpu.sync_copy(acc.at[pl.ds(sid * RPT, RPT), pl.ds(0, gD)],
                            out.at[pl.ds(sid * RPT, RPT), pl.ds(g0, gD)])
            plsc.subcore_barrier()

        for core_id in (0, 1):
            if per_core[core_id]:
                @pl.when(cc == core_id)
                def _(groups=per_core[core_id]):
                    for g0, gD in groups:
                        do_group(g0, gD)

    def run(vals, idx, zeros):
        return pl.kernel(
            body,
            out_type=jax.ShapeDtypeStruct((Npad, D), jnp.float32),
            mesh=mesh,
            compiler_params=pltpu.CompilerParams(use_tc_tiling_on_sc=False),
            scratch_types=[
                pltpu.VMEM((C,), jnp.int32),
                pltpu.VMEM((C, 16), jnp.float32),
                pltpu.VMEM_SHARED((Npad, 16), jnp.float32),
                pltpu.SemaphoreType.DMA,
            ],
        )(vals, idx, zeros)

    return run


def _scatter_add(vals, idx, Npad, zeros):
    E, D = vals.shape
    return _sc_scatter(E, D, Npad, 1000)(vals, idx, zeros)


def _swish(s, beta):
    return s * jax.nn.sigmoid(beta * s)


def _swish_grad(s, beta):
    sb = jax.nn.sigmoid(beta * s)
    return sb + beta * s * sb * (1.0 - sb)


def _edge_fwd_body(hs, vs, xs, xd, Wh, Ws, bs, Wu, Wg, bg, beta, ms_o, mv_o):
    b = beta[0, 0]
    dxv = xd[:, 0:3] - xs[:, 0:3]
    dist2 = jnp.sum(dxv * dxv, axis=1, keepdims=True) + 1e-8
    dist = jnp.sqrt(dist2)
    dirv = dxv / dist
    Vh = []
    for d in range(3):
        mvin = jnp.concatenate([vs[:, 16 * d:16 * d + 16], dirv[:, d:d + 1]], 1)
        Vh.append(jnp.dot(mvin, Wh[:], preferred_element_type=jnp.float32))
    vn2 = Vh[0] * Vh[0] + Vh[1] * Vh[1] + Vh[2] * Vh[2]
    vn = jnp.sqrt(vn2 + 1e-8)
    scat = jnp.concatenate([hs[:], dist, vn], 1)
    slin = jnp.dot(scat, Ws[:], preferred_element_type=jnp.float32) + bs[:]
    m_s = _swish(slin, b)
    gate = jax.nn.sigmoid(jnp.dot(m_s, Wg[:], preferred_element_type=jnp.float32) + bg[:])
    outs = []
    for d in range(3):
        Vu = jnp.dot(Vh[d], Wu[:], preferred_element_type=jnp.float32)
        outs.append(Vu * gate)
    ms_o[:] = m_s
    mv = jnp.concatenate(outs, 1)
    mv_o[:] = jnp.pad(mv, ((0, 0), (0, mv_o.shape[1] - mv.shape[1])))


def _edge_bwd_body(hs, vs, xs, xd, dms_i, dmv_i,
                   Wh, Ws, bs, Wu, Wg, bg, beta,
                   dhs_o, dvs_o, ddxv_o):
    b = beta[0, 0]
    dxv = xd[:, 0:3] - xs[:, 0:3]
    dist2 = jnp.sum(dxv * dxv, axis=1, keepdims=True) + 1e-8
    dist = jnp.sqrt(dist2)
    dirv = dxv / dist
    Vh = []
    for d in range(3):
        mvin = jnp.concatenate([vs[:, 16 * d:16 * d + 16], dirv[:, d:d + 1]], 1)
        Vh.append(jnp.dot(mvin, Wh[:], preferred_element_type=jnp.float32))
    vn2 = Vh[0] * Vh[0] + Vh[1] * Vh[1] + Vh[2] * Vh[2]
    vn = jnp.sqrt(vn2 + 1e-8)
    scat = jnp.concatenate([hs[:], dist, vn], 1)
    slin = jnp.dot(scat, Ws[:], preferred_element_type=jnp.float32) + bs[:]
    m_s = _swish(slin, b)
    g_lin = jnp.dot(m_s, Wg[:], preferred_element_type=jnp.float32) + bg[:]
    gate = jax.nn.sigmoid(g_lin)
    Vu = [jnp.dot(Vh[d], Wu[:], preferred_element_type=jnp.float32) for d in range(3)]
    dmv = [dmv_i[:, 17 * d:17 * d + 17] for d in range(3)]
    dgate = dmv[0] * Vu[0] + dmv[1] * Vu[1] + dmv[2] * Vu[2]
    dg = dgate * gate * (1.0 - gate)
    dms = dms_i[:] + jnp.dot(dg, Wg[:].T, preferred_element_type=jnp.float32)
    dslin = dms * _swish_grad(slin, b)
    dscat = jnp.dot(dslin, Ws[:].T, preferred_element_type=jnp.float32)
    dhs_o[:] = dscat[:, 0:64]
    ddist_s = dscat[:, 64:65]
    dvn = dscat[:, 65:82]
    ddirv = []
    dvs = []
    for d in range(3):
        dVh = (jnp.dot(dmv[d] * gate, Wu[:].T, preferred_element_type=jnp.float32)
               + dvn * Vh[d] / vn)
        dmvin = jnp.dot(dVh, Wh[:].T, preferred_element_type=jnp.float32)
        dvs.append(dmvin[:, 0:16])
        ddirv.append(dmvin[:, 16:17])
    dvs_o[:] = jnp.concatenate(dvs, 1)
    ddirv = jnp.concatenate(ddirv, 1)
    proj = jnp.sum(ddirv * dxv, axis=1, keepdims=True)
    ddist_tot = ddist_s - proj / dist2
    ddxv = ddirv / dist + ddist_tot * dxv / dist
    ddxv_o[:] = jnp.pad(ddxv, ((0, 0), (0, ddxv_o.shape[1] - 3)))


def _upd_fwd_body(h, aggs, v, aggv, Wh, Ws, bs, Wu, Wg, bg, beta, us_o, uv_o):
    b = beta[0, 0]
    Vh = []
    for d in range(3):
        vin = jnp.concatenate([v[:, 16 * d:16 * d + 16], aggv[:, 16 * d:16 * d + 16]], 1)
        Vh.append(jnp.dot(vin, Wh[:], preferred_element_type=jnp.float32))
    vn = jnp.sqrt(Vh[0] * Vh[0] + Vh[1] * Vh[1] + Vh[2] * Vh[2] + 1e-8)
    scat = jnp.concatenate([h[:], aggs[:], vn], 1)
    slin = jnp.dot(scat, Ws[:], preferred_element_type=jnp.float32) + bs[:]
    s_out = _swish(slin, b)
    gate = jax.nn.sigmoid(jnp.dot(s_out, Wg[:], preferred_element_type=jnp.float32) + bg[:])
    us_o[:] = s_out
    uv_o[:] = jnp.concatenate(
        [jnp.dot(Vh[d], Wu[:], preferred_element_type=jnp.float32) * gate for d in range(3)], 1)


def _upd_bwd_body(h, aggs, v, aggv, dus_i, duv_i,
                  Wh, Ws, bs, Wu, Wg, bg, beta,
                  dh_o, daggs_o, dv_o, daggv_o):
    b = beta[0, 0]
    Vh = []
    for d in range(3):
        vin = jnp.concatenate([v[:, 16 * d:16 * d + 16], aggv[:, 16 * d:16 * d + 16]], 1)
        Vh.append(jnp.dot(vin, Wh[:], preferred_element_type=jnp.float32))
    vn = jnp.sqrt(Vh[0] * Vh[0] + Vh[1] * Vh[1] + Vh[2] * Vh[2] + 1e-8)
    scat = jnp.concatenate([h[:], aggs[:], vn], 1)
    slin = jnp.dot(scat, Ws[:], preferred_element_type=jnp.float32) + bs[:]
    s_out = _swish(slin, b)
    g_lin = jnp.dot(s_out, Wg[:], preferred_element_type=jnp.float32) + bg[:]
    gate = jax.nn.sigmoid(g_lin)
    Vu = [jnp.dot(Vh[d], Wu[:], preferred_element_type=jnp.float32) for d in range(3)]
    duv = [duv_i[:, 16 * d:16 * d + 16] for d in range(3)]
    dgate = duv[0] * Vu[0] + duv[1] * Vu[1] + duv[2] * Vu[2]
    dg = dgate * gate * (1.0 - gate)
    ds_out = dus_i[:] + jnp.dot(dg, Wg[:].T, preferred_element_type=jnp.float32)
    dslin = ds_out * _swish_grad(slin, b)
    dscat = jnp.dot(dslin, Ws[:].T, preferred_element_type=jnp.float32)
    dh_o[:] = dscat[:, 0:64]
    daggs_o[:] = dscat[:, 64:128]
    dvn = dscat[:, 128:160]
    dv = []
    daggv = []
    for d in range(3):
        dVh = (jnp.dot(duv[d] * gate, Wu[:].T, preferred_element_type=jnp.float32)
               + dvn * Vh[d] / vn)
        dvin = jnp.dot(dVh, Wh[:].T, preferred_element_type=jnp.float32)
        dv.append(dvin[:, 0:16])
        daggv.append(dvin[:, 16:32])
    dv_o[:] = jnp.concatenate(dv, 1)
    daggv_o[:] = jnp.concatenate(daggv, 1)


def _emb_body(zc, w, bvec, zs_o):
    zlin = jnp.dot(zc[:], w[:], preferred_element_type=jnp.float32) + bvec[:]
    zs_o[:] = zlin * jax.nn.sigmoid(zlin)


def _mlp_body(hs, w1, b1, w2, b2, e_o, dhs_o, *, NPc):
    BGr = hs.shape[0] // NPc
    pooled = jnp.mean(hs[:].reshape(BGr, NPc, hs.shape[1]), axis=1)
    z1 = jnp.dot(pooled, w1[:], preferred_element_type=jnp.float32) + b1[:]
    sg = jax.nn.sigmoid(z1)
    a = z1 * sg
    e_o[:] = jnp.dot(a, w2[:], preferred_element_type=jnp.float32) + b2[:]
    da = jnp.broadcast_to(w2[:].T, (BGr, hs.shape[1]))
    dz1 = da * sg * (1.0 + z1 * (1.0 - sg))
    dpool = jnp.dot(dz1, w1[:].T, preferred_element_type=jnp.float32) * (1.0 / NPc)
    dhs_o[:] = jnp.broadcast_to(dpool[:, None, :], (BGr, NPc, hs.shape[1])).reshape(
        BGr * NPc, hs.shape[1])


def _pick(n, pref):
    return pref if n % pref == 0 else n

def _full(r, c):
    return pl.BlockSpec((r, c), lambda i: (0, 0))


def _blk(r, c):
    return pl.BlockSpec((r, c), lambda i: (i, 0))


def _edge_fwd(hs, vs, xs, xd, p):
    E = hs.shape[0]
    BE = _pick(E, 3200)
    grid = (E // BE,)
    return pl.pallas_call(
        _edge_fwd_body,
        grid=grid,
        in_specs=[_blk(BE, 64), _blk(BE, 48), _blk(BE, 16), _blk(BE, 16),
                  _full(17, 17), _full(82, 64), _full(1, 64),
                  _full(17, 17), _full(64, 17), _full(1, 17),
                  _full(1, 1)],
        out_specs=[_blk(BE, 64), _blk(BE, 56)],
        out_shape=[jax.ShapeDtypeStruct((E, 64), jnp.float32),
                   jax.ShapeDtypeStruct((E, 56), jnp.float32)],
    )(hs, vs, xs, xd, p['Wh'], p['Ws'], p['bs'].reshape(1, -1), p['Wu'], p['Wg'], p['bg'].reshape(1, -1),
      p['beta'].reshape(1, 1))


def _edge_bwd(hs, vs, xs, xd, dms, dmv, p):
    E = hs.shape[0]
    BE = _pick(E, 3200)
    grid = (E // BE,)
    return pl.pallas_call(
        _edge_bwd_body,
        grid=grid,
        in_specs=[_blk(BE, 64), _blk(BE, 48), _blk(BE, 16), _blk(BE, 16),
                  _blk(BE, 64), _blk(BE, 56),
                  _full(17, 17), _full(82, 64), _full(1, 64),
                  _full(17, 17), _full(64, 17), _full(1, 17),
                  _full(1, 1)],
        out_specs=[_blk(BE, 64), _blk(BE, 48), _blk(BE, 16)],
        out_shape=[jax.ShapeDtypeStruct((E, 64), jnp.float32),
                   jax.ShapeDtypeStruct((E, 48), jnp.float32),
                   jax.ShapeDtypeStruct((E, 16), jnp.float32)],
    )(hs, vs, xs, xd, dms, dmv, p['Wh'], p['Ws'], p['bs'].reshape(1, -1), p['Wu'], p['Wg'],
      p['bg'].reshape(1, -1), p['beta'].reshape(1, 1))


def _upd_fwd(h, aggs, v, aggv, p):
    N = h.shape[0]
    BN = _pick(N, 2000)
    grid = (N // BN,)
    return pl.pallas_call(
        _upd_fwd_body,
        grid=grid,
        in_specs=[_blk(BN, 64), _blk(BN, 64), _blk(BN, 48), _blk(BN, 48),
                  _full(32, 32), _full(160, 64), _full(1, 64),
                  _full(32, 16), _full(64, 16), _full(1, 16),
                  _full(1, 1)],
        out_specs=[_blk(BN, 64), _blk(BN, 48)],
        out_shape=[jax.ShapeDtypeStruct((N, 64), jnp.float32),
                   jax.ShapeDtypeStruct((N, 48), jnp.float32)],
    )(h, aggs, v, aggv, p['Wh'], p['Ws'], p['bs'].reshape(1, -1), p['Wu'], p['Wg'], p['bg'].reshape(1, -1),
      p['beta'].reshape(1, 1))


def _upd_bwd(h, aggs, v, aggv, dus, duv, p):
    N = h.shape[0]
    BN = _pick(N, 2000)
    grid = (N // BN,)
    return pl.pallas_call(
        _upd_bwd_body,
        grid=grid,
        in_specs=[_blk(BN, 64), _blk(BN, 64), _blk(BN, 48), _blk(BN, 48),
                  _blk(BN, 64), _blk(BN, 48),
                  _full(32, 32), _full(160, 64), _full(1, 64),
                  _full(32, 16), _full(64, 16), _full(1, 16),
                  _full(1, 1)],
        out_specs=[_blk(BN, 64), _blk(BN, 64), _blk(BN, 48), _blk(BN, 48)],
        out_shape=[jax.ShapeDtypeStruct((N, 64), jnp.float32),
                   jax.ShapeDtypeStruct((N, 64), jnp.float32),
                   jax.ShapeDtypeStruct((N, 48), jnp.float32),
                   jax.ShapeDtypeStruct((N, 48), jnp.float32)],
    )(h, aggs, v, aggv, dus, duv, p['Wh'], p['Ws'], p['bs'].reshape(1, -1), p['Wu'], p['Wg'],
      p['bg'].reshape(1, -1), p['beta'].reshape(1, 1))


def _emb(zc, w, bvec):
    N, K = zc.shape
    BN = _pick(N, 2000)
    H = w.shape[1]
    return pl.pallas_call(
        _emb_body,
        grid=(N // BN,),
        in_specs=[_blk(BN, K), _full(K, H), _full(1, H)],
        out_specs=_blk(BN, H),
        out_shape=jax.ShapeDtypeStruct((N, H), jnp.float32),
    )(zc, w, bvec)


def _mlp_seed(hs, w1, b1, w2, b2, NPc):
    N, H = hs.shape
    G = N // NPc
    BG = _pick(G, 400)
    return pl.pallas_call(
        functools.partial(_mlp_body, NPc=NPc),
        grid=(G // BG,),
        in_specs=[_blk(BG * NPc, H), _full(H, H), _full(1, H),
                  _full(H, 1), _full(1, 1)],
        out_specs=[_blk(BG, 1), _blk(BG * NPc, H)],
        out_shape=[jax.ShapeDtypeStruct((G, 1), jnp.float32),
                   jax.ShapeDtypeStruct((N, H), jnp.float32)],
    )(hs, w1, b1, w2, b2)


def _pad16(x3):
    return jnp.pad(x3, ((0, 0), (0, 13)))


def kernel(t, h, x, edge_index, params):
    N = h.shape[0]
    E = edge_index.shape[1]
    G = t.shape[0]
    NPc = N // G
    Npad = ((N + 127) // 128) * 128
    src = edge_index[0]
    dst = edge_index[1]
    zeros_pad = jnp.zeros((Npad, 16), jnp.float32)

    ones_e = jnp.ones((E, 16), jnp.float32)
    cnt = _scatter_add(ones_e, dst, Npad, zeros_pad)[:N, 0]
    inv_c = (1.0 / jnp.maximum(cnt, 1.0))[:, None]

    ts = jnp.repeat(t, NPc).reshape(-1, 1)
    zs = _emb(jnp.concatenate([h, ts], axis=1), params['emb_w'], params['emb_b'].reshape(1, -1))

    hcur = zs
    vcur = jnp.zeros((N, 48), jnp.float32)
    xcur = x
    res = []
    for li in range(len(params['layers'])):
        lp = params['layers'][li]
        xt = _pad16(xcur)
        hs_e = _gather(hcur, src)
        vs_e = _gather(vcur, src)
        xs_e = _gather(xt, src)
        xd_e = _gather(xt, dst)
        ms_e, mv_e = _edge_fwd(hs_e, vs_e, xs_e, xd_e, lp['msg'])
        aggs = _scatter_add(ms_e, dst, Npad, zeros_pad)[:N] * inv_c
        agg56 = _scatter_add(mv_e, dst, Npad, zeros_pad)[:N] * inv_c
        a3 = agg56[:, :51].reshape(N, 3, 17)
        aggv = a3[:, :, :16].reshape(N, 48)
        aggx = a3[:, :, 16]
        us, uv = _upd_fwd(hcur, aggs, vcur, aggv, lp['upd'])
        tnh = jnp.tanh(aggx)
        res.append(dict(h=hcur, v=vcur, x=xcur, hs_e=hs_e, vs_e=vs_e,
                        xs_e=xs_e, xd_e=xd_e, aggs=aggs, aggv=aggv, tnh=tnh))
        hcur = hcur + us
        vcur = vcur + uv
        xcur = xcur + CR * tnh

    e, dh = _mlp_seed(hcur, params['out_w1'], params['out_b1'].reshape(1, -1),
                      params['out_w2'], params['out_b2'].reshape(1, -1), NPc)

    dv = jnp.zeros((N, 48), jnp.float32)
    dx = jnp.zeros((N, 3), jnp.float32)
    for li in range(len(params['layers']) - 1, -1, -1):
        lp = params['layers'][li]
        r = res[li]
        daggx = dx * CR * (1.0 - r['tnh'] * r['tnh'])
        dh_u, daggs, dv_u, daggv = _upd_bwd(r['h'], r['aggs'], r['v'], r['aggv'],
                                            dh, dv, lp['upd'])
        dh = dh + dh_u
        dv = dv + dv_u
        d3 = jnp.concatenate([daggv.reshape(N, 3, 16), daggx[:, :, None]], axis=2)
        dagg56 = jnp.pad(d3.reshape(N, 51), ((0, 0), (0, 5))) * inv_c
        dms_e = _gather(daggs * inv_c, dst)
        dmv_e = _gather(dagg56, dst)
        dhs_e, dvs_e, ddxv_e = _edge_bwd(r['hs_e'], r['vs_e'], r['xs_e'], r['xd_e'],
                                         dms_e, dmv_e, lp['msg'])
        dh = dh + _scatter_add(dhs_e, src, Npad, zeros_pad)[:N]
        dv = dv + _scatter_add(dvs_e, src, Npad, zeros_pad)[:N]
        sc_d = _scatter_add(ddxv_e, dst, Npad, zeros_pad)[:N, :3]
        sc_s = _scatter_add(ddxv_e, src, Npad, zeros_pad)[:N, :3]
        dx = dx + sc_d - sc_s

    return (dx, e)
